# baseline (speedup 1.0000x reference)
import jax
import jax.numpy as jnp
from jax import lax
from jax.experimental import pallas as pl
from jax.experimental.pallas import tpu as pltpu

NCLS = 80
H = 52
W = 52
HW = H * W
A = 3
B = 8
N = A * HW
NP = 8192
MAXD = 300
NMS_T = 0.7
SCORE_T = 0.05
NEG = float("-inf")
BIGI = 1 << 30
T = 8


def _decode_kernel(x_ref, anc_ref, o_ref):
    a = pl.program_id(1)
    iota = lax.broadcasted_iota(jnp.int32, (1, HW), 1)
    wf = (iota % W).astype(jnp.float32)
    hf = (iota // W).astype(jnp.float32)
    sx = jax.nn.sigmoid(x_ref[0, 0, 0:1, :])
    sy = jax.nn.sigmoid(x_ref[0, 0, 1:2, :])
    bx = (sx + wf) / float(W)
    by = (sy + hf) / float(H)
    aw = anc_ref[a, 0]
    ah = anc_ref[a, 1]
    bw = jnp.clip(jnp.exp(x_ref[0, 0, 2:3, :]) * aw, 0.0, 2.0)
    bh = jnp.clip(jnp.exp(x_ref[0, 0, 3:4, :]) * ah, 0.0, 2.0)
    x1 = bx - 0.5 * bw
    y1 = by - 0.5 * bh
    x2 = x1 + bw
    y2 = y1 + bh
    x1c = jnp.clip(x1, 0.0, 1.0)
    y1c = jnp.clip(y1, 0.0, 1.0)
    x2c = jnp.clip(x2, 0.0, 1.0)
    y2c = jnp.clip(y2, 0.0, 1.0)
    bo = jax.nn.sigmoid(x_ref[0, 0, 4:5, :])
    scls = jax.nn.sigmoid(x_ref[0, 0, 5:85, :])
    mx = jnp.max(scls, axis=0, keepdims=True)
    c_iota = lax.broadcasted_iota(jnp.int32, (NCLS, HW), 0)
    lab = jnp.min(
        jnp.where(scls == mx, c_iota, NCLS), axis=0, keepdims=True
    ).astype(jnp.float32)
    score = mx * bo
    valid = (bo >= 0.5) & (score >= SCORE_T)
    msc = jnp.where(valid, score, NEG)
    o_ref[0, 0] = jnp.concatenate([x1c, y1c, x2c, y2c, msc, lab], axis=0)


def _pair_iou(ax1, ay1, ax2, ay2, aar, bx1, by1, bx2, by2, bar):
    xx1 = jnp.maximum(ax1, bx1)
    yy1 = jnp.maximum(ay1, by1)
    xx2 = jnp.minimum(ax2, bx2)
    yy2 = jnp.minimum(ay2, by2)
    inter = jnp.maximum(xx2 - xx1, 0.0) * jnp.maximum(yy2 - yy1, 0.0)
    return inter / (aar + bar - inter + 1e-12)


def _nms_kernel(x1_ref, y1_ref, x2_ref, y2_ref, s_in_ref, lb_ref,
                ds_ref, dx1_ref, dy1_ref, dx2_ref, dy2_ref, dl_ref, nd_ref,
                s_ref, ar_ref, mv_ref, k_ref, done_ref):
    s0 = s_in_ref[...]
    s_ref[...] = s0
    x1v = x1_ref[...]
    y1v = y1_ref[...]
    x2v = x2_ref[...]
    y2v = y2_ref[...]
    ar_ref[...] = (x2v - x1v) * (y2v - y1v)
    mv_ref[...] = jnp.max(s0, axis=1, keepdims=True)
    k_ref[...] = jnp.zeros((B, 1), jnp.int32)
    done_ref[...] = jnp.zeros((B, 1), jnp.int32)

    kiota = lax.broadcasted_iota(jnp.int32, (B, MAXD), 1)
    zf = jnp.zeros((B, MAXD), jnp.float32)
    ds_ref[...] = zf
    dx1_ref[...] = zf
    dy1_ref[...] = zf
    dx2_ref[...] = zf
    dy2_ref[...] = zf
    dl_ref[...] = jnp.zeros((B, MAXD), jnp.int32)

    def cond(go):
        return go != 0

    def body(go):
        del go
        k = k_ref[...]
        done = done_ref[...] != 0
        has = mv_ref[...] > NEG
        act = jnp.logical_not(done) & has

        s = s_ref[...]
        ridx = lax.broadcasted_iota(jnp.int32, (B, NP), 1)
        x1 = x1_ref[...]
        y1 = y1_ref[...]
        x2 = x2_ref[...]
        y2 = y2_ref[...]
        ar = ar_ref[...]
        lb = lb_ref[...]

        ms = []
        rmins = []
        s_work = s
        for i in range(T):
            if i == 0:
                m_i = mv_ref[...]
            else:
                m_i = jnp.max(s_work, axis=1, keepdims=True)
            rmin_i = jnp.min(jnp.where(s_work == m_i, ridx, BIGI), axis=1,
                             keepdims=True)
            s_work = jnp.where(ridx == rmin_i, NEG, s_work)
            ms.append(m_i)
            rmins.append(rmin_i)

        pay = []
        for i in range(T):
            w = ridx == rmins[i]
            px1 = jnp.sum(jnp.where(w, x1, 0.0), axis=1, keepdims=True)
            py1 = jnp.sum(jnp.where(w, y1, 0.0), axis=1, keepdims=True)
            px2 = jnp.sum(jnp.where(w, x2, 0.0), axis=1, keepdims=True)
            py2 = jnp.sum(jnp.where(w, y2, 0.0), axis=1, keepdims=True)
            plb = jnp.sum(jnp.where(w, lb, 0.0), axis=1, keepdims=True)
            par = (px2 - px1) * (py2 - py1)
            pay.append((px1, py1, px2, py2, par, plb))

        emit = []
        slots = []
        k_run = k
        for i in range(T):
            killed = jnp.zeros((B, 1), jnp.bool_)
            for jj in range(i):
                iou_ji = _pair_iou(*pay[jj][:5], *pay[i][:5])
                killed = killed | (emit[jj] & (iou_ji > NMS_T))
            e_i = act & (ms[i] > NEG) & jnp.logical_not(killed) \
                & (k_run < MAXD)
            emit.append(e_i)
            slots.append(k_run)
            k_run = k_run + e_i.astype(jnp.int32)

        kill = None
        for i in range(T):
            iou_i = _pair_iou(pay[i][0], pay[i][1], pay[i][2], pay[i][3],
                              pay[i][4], x1, y1, x2, y2, ar)
            k_i = emit[i] & (iou_i > NMS_T)
            kill = k_i if kill is None else (kill | k_i)
        news = jnp.where(act, jnp.where(kill, NEG, s_work), s)
        s_ref[...] = news
        mv_ref[...] = jnp.max(news, axis=1, keepdims=True)

        zm = jnp.zeros((B, MAXD), jnp.float32)
        a_ds = zm
        a_x1 = zm
        a_y1 = zm
        a_x2 = zm
        a_y2 = zm
        a_lb = jnp.zeros((B, MAXD), jnp.int32)
        for i in range(T):
            oh = (kiota == slots[i]) & emit[i]
            px1, py1, px2, py2, par, plb = pay[i]
            a_ds = a_ds + jnp.where(oh, ms[i], 0.0)
            a_x1 = a_x1 + jnp.where(oh, px1, 0.0)
            a_y1 = a_y1 + jnp.where(oh, py1, 0.0)
            a_x2 = a_x2 + jnp.where(oh, px2, 0.0)
            a_y2 = a_y2 + jnp.where(oh, py2, 0.0)
            a_lb = a_lb + jnp.where(oh, plb.astype(jnp.int32), 0)
        ds_ref[...] = ds_ref[...] + a_ds
        dx1_ref[...] = dx1_ref[...] + a_x1
        dy1_ref[...] = dy1_ref[...] + a_y1
        dx2_ref[...] = dx2_ref[...] + a_x2
        dy2_ref[...] = dy2_ref[...] + a_y2
        dl_ref[...] = dl_ref[...] + a_lb

        done_new = done | (k_run >= MAXD) | jnp.logical_not(has)
        k_ref[...] = k_run
        done_ref[...] = done_new.astype(jnp.int32)
        n_done = jnp.sum(done_new.astype(jnp.int32))
        return jnp.where(n_done < B, jnp.int32(1), jnp.int32(0))

    lax.while_loop(cond, body, jnp.int32(1))
    nd_ref[...] = k_ref[...]


def _run(x, anchors, interpret=False):
    xr = x.reshape(B, A, 5 + NCLS, HW)
    cand = pl.pallas_call(
        _decode_kernel,
        grid=(B, A),
        in_specs=[
            pl.BlockSpec((1, 1, 5 + NCLS, HW), lambda b, a: (b, a, 0, 0)),
            pl.BlockSpec(memory_space=pltpu.SMEM),
        ],
        out_specs=pl.BlockSpec((1, 1, 6, HW), lambda b, a: (b, a, 0, 0)),
        out_shape=jax.ShapeDtypeStruct((B, A, 6, HW), jnp.float32),
        interpret=interpret,
    )(xr, anchors)

    fields = cand.transpose(0, 2, 3, 1).reshape(B, 6, N)
    pad = jnp.zeros((B, 6, NP - N), jnp.float32)
    pad = pad.at[:, 4, :].set(NEG)
    fields = jnp.concatenate([fields, pad], axis=2)
    f_x1, f_y1, f_x2, f_y2, f_s, f_lb = (fields[:, i] for i in range(6))

    outs = pl.pallas_call(
        _nms_kernel,
        in_specs=[pl.BlockSpec(memory_space=pltpu.VMEM)] * 6,
        out_specs=[pl.BlockSpec(memory_space=pltpu.VMEM)] * 7,
        out_shape=[
            jax.ShapeDtypeStruct((B, MAXD), jnp.float32),
            jax.ShapeDtypeStruct((B, MAXD), jnp.float32),
            jax.ShapeDtypeStruct((B, MAXD), jnp.float32),
            jax.ShapeDtypeStruct((B, MAXD), jnp.float32),
            jax.ShapeDtypeStruct((B, MAXD), jnp.float32),
            jax.ShapeDtypeStruct((B, MAXD), jnp.int32),
            jax.ShapeDtypeStruct((B, 1), jnp.int32),
        ],
        scratch_shapes=[
            pltpu.VMEM((B, NP), jnp.float32),
            pltpu.VMEM((B, NP), jnp.float32),
            pltpu.VMEM((B, 1), jnp.float32),
            pltpu.VMEM((B, 1), jnp.int32),
            pltpu.VMEM((B, 1), jnp.int32),
        ],
        interpret=interpret,
    )(f_x1, f_y1, f_x2, f_y2, f_s, f_lb)
    ds, dx1, dy1, dx2, dy2, dl, nd = outs
    det_boxes = jnp.stack([dx1, dy1, dx2, dy2], axis=-1)
    return det_boxes, ds, dl, nd.reshape(B)


def kernel(x, anchors):
    return _run(x, anchors, interpret=False)


# speedup vs baseline: 1.5565x; 1.5565x over previous
import jax
import jax.numpy as jnp
from jax import lax
from jax.experimental import pallas as pl
from jax.experimental.pallas import tpu as pltpu

NCLS = 80
H = 52
W = 52
HW = H * W
A = 3
B = 8
N = A * HW
MAXD = 300
NMS_T = 0.7
SCORE_T = 0.05
NEG = float("-inf")
BIGI = 1 << 30
T = 8


def _pair_iou(ax1, ay1, ax2, ay2, aar, bx1, by1, bx2, by2, bar):
    xx1 = jnp.maximum(ax1, bx1)
    yy1 = jnp.maximum(ay1, by1)
    xx2 = jnp.minimum(ax2, bx2)
    yy2 = jnp.minimum(ay2, by2)
    inter = jnp.maximum(xx2 - xx1, 0.0) * jnp.maximum(yy2 - yy1, 0.0)
    return inter / (aar + bar - inter + 1e-12)


def _fused_kernel(x_ref, anc_ref,
                  ds_ref, dx1_ref, dy1_ref, dx2_ref, dy2_ref, dl_ref, nd_ref,
                  fx1_ref, fy1_ref, fx2_ref, fy2_ref, fs_ref, flb_ref,
                  ridx_ref, mv_ref, k_ref, done_ref):
    iota = lax.broadcasted_iota(jnp.int32, (1, HW), 1)
    wf = (iota % W).astype(jnp.float32)
    hf = (iota // W).astype(jnp.float32)
    c_iota = lax.broadcasted_iota(jnp.int32, (NCLS, HW), 0)
    for b in range(B):
        for a in range(A):
            r0 = a * (5 + NCLS)
            sx = jax.nn.sigmoid(x_ref[b, r0 + 0:r0 + 1, :])
            sy = jax.nn.sigmoid(x_ref[b, r0 + 1:r0 + 2, :])
            bx = (sx + wf) / float(W)
            by = (sy + hf) / float(H)
            aw = anc_ref[a, 0]
            ah = anc_ref[a, 1]
            bw = jnp.clip(jnp.exp(x_ref[b, r0 + 2:r0 + 3, :]) * aw, 0.0, 2.0)
            bh = jnp.clip(jnp.exp(x_ref[b, r0 + 3:r0 + 4, :]) * ah, 0.0, 2.0)
            x1 = bx - 0.5 * bw
            y1 = by - 0.5 * bh
            x2 = x1 + bw
            y2 = y1 + bh
            bo = jax.nn.sigmoid(x_ref[b, r0 + 4:r0 + 5, :])
            scls = jax.nn.sigmoid(x_ref[b, r0 + 5:r0 + 85, :])
            mx = jnp.max(scls, axis=0, keepdims=True)
            lab = jnp.min(
                jnp.where(scls == mx, c_iota, NCLS), axis=0, keepdims=True
            ).astype(jnp.float32)
            score = mx * bo
            valid = (bo >= 0.5) & (score >= SCORE_T)
            msc = jnp.where(valid, score, NEG)
            sl = slice(b, b + 1), slice(a * HW, (a + 1) * HW)
            fx1_ref[sl] = jnp.clip(x1, 0.0, 1.0)
            fy1_ref[sl] = jnp.clip(y1, 0.0, 1.0)
            fx2_ref[sl] = jnp.clip(x2, 0.0, 1.0)
            fy2_ref[sl] = jnp.clip(y2, 0.0, 1.0)
            fs_ref[sl] = msc
            flb_ref[sl] = lab

    jj = lax.broadcasted_iota(jnp.int32, (B, N), 1)
    aidx = jj // HW
    ridx_ref[...] = (jj - aidx * HW) * A + aidx
    mv_ref[...] = jnp.max(fs_ref[...], axis=1, keepdims=True)
    k_ref[...] = jnp.zeros((B, 1), jnp.int32)
    done_ref[...] = jnp.zeros((B, 1), jnp.int32)

    kiota = lax.broadcasted_iota(jnp.int32, (B, MAXD), 1)
    zf = jnp.zeros((B, MAXD), jnp.float32)
    ds_ref[...] = zf
    dx1_ref[...] = zf
    dy1_ref[...] = zf
    dx2_ref[...] = zf
    dy2_ref[...] = zf
    dl_ref[...] = jnp.zeros((B, MAXD), jnp.int32)

    def cond(go):
        return go != 0

    def body(go):
        del go
        k = k_ref[...]
        done = done_ref[...] != 0
        has = mv_ref[...] > NEG
        act = jnp.logical_not(done) & has

        s = fs_ref[...]
        ridx = ridx_ref[...]
        lane = lax.broadcasted_iota(jnp.int32, (B, N), 1)
        x1 = fx1_ref[...]
        y1 = fy1_ref[...]
        x2 = fx2_ref[...]
        y2 = fy2_ref[...]
        lb = flb_ref[...]
        ar = (x2 - x1) * (y2 - y1)

        ms = []
        lanes = []
        s_work = s
        for i in range(T):
            if i == 0:
                m_i = mv_ref[...]
            else:
                m_i = jnp.max(s_work, axis=1, keepdims=True)
            rmin_i = jnp.min(jnp.where(s_work == m_i, ridx, BIGI), axis=1,
                             keepdims=True)
            lane_i = (rmin_i % A) * HW + rmin_i // A
            s_work = jnp.where(lane == lane_i, NEG, s_work)
            ms.append(m_i)
            lanes.append(lane_i)

        pay = []
        for i in range(T):
            w = lane == lanes[i]
            px1 = jnp.sum(jnp.where(w, x1, 0.0), axis=1, keepdims=True)
            py1 = jnp.sum(jnp.where(w, y1, 0.0), axis=1, keepdims=True)
            px2 = jnp.sum(jnp.where(w, x2, 0.0), axis=1, keepdims=True)
            py2 = jnp.sum(jnp.where(w, y2, 0.0), axis=1, keepdims=True)
            plb = jnp.sum(jnp.where(w, lb, 0.0), axis=1, keepdims=True)
            par = (px2 - px1) * (py2 - py1)
            pay.append((px1, py1, px2, py2, par, plb))

        emit = []
        slots = []
        k_run = k
        for i in range(T):
            killed = jnp.zeros((B, 1), jnp.bool_)
            for jx in range(i):
                iou_ji = _pair_iou(*pay[jx][:5], *pay[i][:5])
                killed = killed | (emit[jx] & (iou_ji > NMS_T))
            e_i = act & (ms[i] > NEG) & jnp.logical_not(killed) \
                & (k_run < MAXD)
            emit.append(e_i)
            slots.append(k_run)
            k_run = k_run + e_i.astype(jnp.int32)

        kill = None
        for i in range(T):
            iou_i = _pair_iou(pay[i][0], pay[i][1], pay[i][2], pay[i][3],
                              pay[i][4], x1, y1, x2, y2, ar)
            k_i = emit[i] & (iou_i > NMS_T)
            kill = k_i if kill is None else (kill | k_i)
        news = jnp.where(act, jnp.where(kill, NEG, s_work), s)
        fs_ref[...] = news
        mv_ref[...] = jnp.max(news, axis=1, keepdims=True)

        zm = jnp.zeros((B, MAXD), jnp.float32)
        a_ds = zm
        a_x1 = zm
        a_y1 = zm
        a_x2 = zm
        a_y2 = zm
        a_lb = jnp.zeros((B, MAXD), jnp.int32)
        for i in range(T):
            oh = (kiota == slots[i]) & emit[i]
            px1, py1, px2, py2, par, plb = pay[i]
            a_ds = a_ds + jnp.where(oh, ms[i], 0.0)
            a_x1 = a_x1 + jnp.where(oh, px1, 0.0)
            a_y1 = a_y1 + jnp.where(oh, py1, 0.0)
            a_x2 = a_x2 + jnp.where(oh, px2, 0.0)
            a_y2 = a_y2 + jnp.where(oh, py2, 0.0)
            a_lb = a_lb + jnp.where(oh, plb.astype(jnp.int32), 0)
        ds_ref[...] = ds_ref[...] + a_ds
        dx1_ref[...] = dx1_ref[...] + a_x1
        dy1_ref[...] = dy1_ref[...] + a_y1
        dx2_ref[...] = dx2_ref[...] + a_x2
        dy2_ref[...] = dy2_ref[...] + a_y2
        dl_ref[...] = dl_ref[...] + a_lb

        done_new = done | (k_run >= MAXD) | jnp.logical_not(has)
        k_ref[...] = k_run
        done_ref[...] = done_new.astype(jnp.int32)
        n_done = jnp.sum(done_new.astype(jnp.int32))
        return jnp.where(n_done < B, jnp.int32(1), jnp.int32(0))

    lax.while_loop(cond, body, jnp.int32(1))
    nd_ref[...] = k_ref[...]


def _run(x, anchors, interpret=False):
    xr = x.reshape(B, A * (5 + NCLS), HW)
    outs = pl.pallas_call(
        _fused_kernel,
        in_specs=[
            pl.BlockSpec(memory_space=pltpu.VMEM),
            pl.BlockSpec(memory_space=pltpu.SMEM),
        ],
        out_specs=[pl.BlockSpec(memory_space=pltpu.VMEM)] * 7,
        out_shape=[
            jax.ShapeDtypeStruct((B, MAXD), jnp.float32),
            jax.ShapeDtypeStruct((B, MAXD), jnp.float32),
            jax.ShapeDtypeStruct((B, MAXD), jnp.float32),
            jax.ShapeDtypeStruct((B, MAXD), jnp.float32),
            jax.ShapeDtypeStruct((B, MAXD), jnp.float32),
            jax.ShapeDtypeStruct((B, MAXD), jnp.int32),
            jax.ShapeDtypeStruct((B, 1), jnp.int32),
        ],
        scratch_shapes=[
            pltpu.VMEM((B, N), jnp.float32),
            pltpu.VMEM((B, N), jnp.float32),
            pltpu.VMEM((B, N), jnp.float32),
            pltpu.VMEM((B, N), jnp.float32),
            pltpu.VMEM((B, N), jnp.float32),
            pltpu.VMEM((B, N), jnp.float32),
            pltpu.VMEM((B, N), jnp.int32),
            pltpu.VMEM((B, 1), jnp.float32),
            pltpu.VMEM((B, 1), jnp.int32),
            pltpu.VMEM((B, 1), jnp.int32),
        ],
        interpret=interpret,
    )(xr, anchors)
    ds, dx1, dy1, dx2, dy2, dl, nd = outs
    det_boxes = jnp.stack([dx1, dy1, dx2, dy2], axis=-1)
    return det_boxes, ds, dl, nd.reshape(B)


def kernel(x, anchors):
    return _run(x, anchors, interpret=False)
